# Initial kernel scaffold; baseline (speedup 1.0000x reference)
import jax
import jax.numpy as jnp
from jax.experimental import pallas as pl


def kernel(x, edge_index, W0, b0, W1, b1, W2, b2, W3, b3, Wv, bv, Wp, bp):
    raise NotImplementedError("write your pallas kernel here")


# speedup vs baseline: 65.0434x; 65.0434x over previous
import functools

import jax
import jax.numpy as jnp
from jax import lax
from jax.experimental import pallas as pl
from jax.experimental.pallas import tpu as pltpu
from jax.experimental.pallas import tpu_sc as plsc

NC = 2
NS = 16
NW = NC * NS
LN = 128
CH = 8
HW = 8


def _sc_mesh():
    return plsc.VectorSubcoreMesh(
        core_axis_name="c", subcore_axis_name="s", num_cores=NC, num_subcores=NS
    )


_SC_PARAMS = pltpu.CompilerParams(use_tc_tiling_on_sc=False)


def _make_deg(n_pad, rows_total):
    rows_per_tile = rows_total // NW
    n_iter = rows_per_tile // CH
    zr = n_pad // NS
    zk = zr // LN

    @functools.partial(
        pl.kernel,
        out_type=jax.ShapeDtypeStruct((NC, n_pad, HW), jnp.float32),
        mesh=_sc_mesh(),
        compiler_params=_SC_PARAMS,
        scratch_types=[
            pltpu.VMEM_SHARED((n_pad, HW), jnp.float32),
            pltpu.VMEM((CH, LN), jnp.int32),
            pltpu.VMEM((LN, HW), jnp.float32),
            pltpu.VMEM((LN, HW), jnp.float32),
            pltpu.SemaphoreType.DMA,
            pltpu.SemaphoreType.DMA,
        ],
    )
    def deg(dst_hbm, ones_hbm, zeros_hbm, acc_out,
            acc_sh, dst_v, ones_v, zbuf, sem_i, sem_s):
        c = lax.axis_index("c")
        s = lax.axis_index("s")
        pltpu.sync_copy(zeros_hbm, zbuf)
        pltpu.sync_copy(ones_hbm, ones_v)

        def zero_body(k, carry):
            pltpu.sync_copy(zbuf, acc_sh.at[pl.ds(s * zr + k * LN, LN)])
            return carry

        lax.fori_loop(0, zk, zero_body, 0)
        plsc.subcore_barrier()

        base = (c * NS + s) * rows_per_tile

        def body(it, carry):
            pltpu.async_copy(dst_hbm.at[pl.ds(base + it * CH, CH)], dst_v, sem_i).wait()
            descs = [
                pltpu.async_copy(ones_v, acc_sh.at[dst_v.at[j]], sem_s, add=True)
                for j in range(CH)
            ]
            for d in descs:
                d.wait()
            return carry

        lax.fori_loop(0, n_iter, body, 0)
        plsc.subcore_barrier()
        pltpu.sync_copy(acc_sh.at[pl.ds(s * zr, zr)], acc_out.at[c, pl.ds(s * zr, zr)])

    return deg


def _make_agg(n_pad, rows_total):
    rows_per_tile = rows_total // NW
    n_iter = rows_per_tile // CH
    zr = n_pad // NS
    zk = zr // LN

    @functools.partial(
        pl.kernel,
        out_type=jax.ShapeDtypeStruct((NC, n_pad, HW), jnp.float32),
        mesh=_sc_mesh(),
        compiler_params=_SC_PARAMS,
        scratch_types=[
            pltpu.VMEM_SHARED((n_pad, HW), jnp.float32),
            pltpu.VMEM((CH, LN), jnp.int32),
            pltpu.VMEM((CH, LN), jnp.int32),
            pltpu.VMEM((CH * LN, HW), jnp.float32),
            pltpu.VMEM((LN, HW), jnp.float32),
            pltpu.SemaphoreType.DMA,
            pltpu.SemaphoreType.DMA,
            pltpu.SemaphoreType.DMA,
        ],
    )
    def agg(src_hbm, dst_hbm, xs_hbm, zeros_hbm, acc_out,
            acc_sh, src_v, dst_v, rows_v, zbuf, sem_i, sem_g, sem_s):
        c = lax.axis_index("c")
        s = lax.axis_index("s")
        pltpu.sync_copy(zeros_hbm, zbuf)

        def zero_body(k, carry):
            pltpu.sync_copy(zbuf, acc_sh.at[pl.ds(s * zr + k * LN, LN)])
            return carry

        lax.fori_loop(0, zk, zero_body, 0)
        plsc.subcore_barrier()

        base = (c * NS + s) * rows_per_tile

        def body(it, carry):
            row0 = base + it * CH
            d1 = pltpu.async_copy(src_hbm.at[pl.ds(row0, CH)], src_v, sem_i)
            d2 = pltpu.async_copy(dst_hbm.at[pl.ds(row0, CH)], dst_v, sem_i)
            d1.wait()
            d2.wait()
            descs = [
                pltpu.async_copy(
                    xs_hbm.at[src_v.at[j]], rows_v.at[pl.ds(j * LN, LN)], sem_g
                )
                for j in range(CH)
            ]
            for d in descs:
                d.wait()
            descs = [
                pltpu.async_copy(
                    rows_v.at[pl.ds(j * LN, LN)], acc_sh.at[dst_v.at[j]], sem_s,
                    add=True,
                )
                for j in range(CH)
            ]
            for d in descs:
                d.wait()
            return carry

        lax.fori_loop(0, n_iter, body, 0)
        plsc.subcore_barrier()
        pltpu.sync_copy(acc_sh.at[pl.ds(s * zr, zr)], acc_out.at[c, pl.ds(s * zr, zr)])

    return agg


_TC_R = 2048


def _dot(a, b):
    return lax.dot_general(a, b, (((1,), (0,)), ((), ())),
                           precision=lax.Precision.DEFAULT,
                           preferred_element_type=jnp.float32)


def _elu(v):
    return jnp.where(v > 0.0, v, jnp.exp(jnp.minimum(v, 0.0)) - 1.0)


def _prep_call(n, deg0, deg1, xp, w0):
    n_pad, ndim = xp.shape
    g = n_pad // _TC_R

    def body(d0_ref, d1_ref, x_ref, w0_ref, dis_ref, xs_ref):
        i = pl.program_id(0)
        rows = i * _TC_R + lax.broadcasted_iota(jnp.int32, (_TC_R, 1), 0)
        deg = d0_ref[...] + d1_ref[...] + 1.0
        dis = jnp.where(rows < n, lax.rsqrt(deg), 0.0)
        dis_ref[...] = dis
        xs_ref[...] = dis * _dot(x_ref[...], w0_ref[...])

    col = lambda i: (i, 0)
    fixed = lambda i: (0, 0)
    return pl.pallas_call(
        body,
        grid=(g,),
        in_specs=[
            pl.BlockSpec((_TC_R, 1), col),
            pl.BlockSpec((_TC_R, 1), col),
            pl.BlockSpec((_TC_R, ndim), col),
            pl.BlockSpec((ndim, HW), fixed),
        ],
        out_specs=[
            pl.BlockSpec((_TC_R, 1), col),
            pl.BlockSpec((_TC_R, HW), col),
        ],
        out_shape=[
            jax.ShapeDtypeStruct((n_pad, 1), jnp.float32),
            jax.ShapeDtypeStruct((n_pad, HW), jnp.float32),
        ],
    )(deg0, deg1, xp, w0)


def _dense_call(acc0, acc1, xs, dis, b_row, w_next):
    n_pad = xs.shape[0]
    g = n_pad // _TC_R

    def body(a0_ref, a1_ref, xs_ref, dis_ref, b_ref, w_ref, out_ref):
        dis = dis_ref[...]
        h = _elu(dis * (a0_ref[...] + a1_ref[...] + xs_ref[...]) + b_ref[...])
        out_ref[...] = dis * _dot(h, w_ref[...])

    col = lambda i: (i, 0)
    fixed = lambda i: (0, 0)
    return pl.pallas_call(
        body,
        grid=(g,),
        in_specs=[
            pl.BlockSpec((_TC_R, HW), col),
            pl.BlockSpec((_TC_R, HW), col),
            pl.BlockSpec((_TC_R, HW), col),
            pl.BlockSpec((_TC_R, 1), col),
            pl.BlockSpec((1, HW), fixed),
            pl.BlockSpec((HW, HW), fixed),
        ],
        out_specs=pl.BlockSpec((_TC_R, HW), col),
        out_shape=jax.ShapeDtypeStruct((n_pad, HW), jnp.float32),
    )(acc0, acc1, xs, dis, b_row, w_next)


def _head_call(n, acc0, acc1, xs, dis, bp, bv):
    n_pad = xs.shape[0]
    g = n_pad // _TC_R

    def body(a0_ref, a1_ref, xs_ref, dis_ref, bp_ref, bv_ref,
             proba_ref, value_ref, vsum):
        i = pl.program_id(0)
        out = dis_ref[...] * (a0_ref[...] + a1_ref[...] + xs_ref[...])
        proba_ref[...] = out[:, 0:1] + bp_ref[...]

        @pl.when(i == 0)
        def _():
            vsum[...] = jnp.zeros_like(vsum)

        vsum[...] += jnp.sum(out[:, 1:2], axis=0, keepdims=True)

        @pl.when(i == g - 1)
        def _():
            value_ref[...] = vsum[...] * (1.0 / n) + bv_ref[...]

    col = lambda i: (i, 0)
    fixed = lambda i: (0, 0)
    return pl.pallas_call(
        body,
        grid=(g,),
        in_specs=[
            pl.BlockSpec((_TC_R, HW), col),
            pl.BlockSpec((_TC_R, HW), col),
            pl.BlockSpec((_TC_R, HW), col),
            pl.BlockSpec((_TC_R, 1), col),
            pl.BlockSpec((1, 1), fixed),
            pl.BlockSpec((1, 1), fixed),
        ],
        out_specs=[
            pl.BlockSpec((_TC_R, 1), col),
            pl.BlockSpec((1, 1), fixed),
        ],
        out_shape=[
            jax.ShapeDtypeStruct((n_pad, 1), jnp.float32),
            jax.ShapeDtypeStruct((1, 1), jnp.float32),
        ],
        scratch_shapes=[pltpu.VMEM((1, 1), jnp.float32)],
    )(acc0, acc1, xs, dis, bp, bv)


def _ceil_to(v, m):
    return -(-v // m) * m


def kernel(x, edge_index, W0, b0, W1, b1, W2, b2, W3, b3, Wv, bv, Wp, bp):
    n, ndim = x.shape
    e = edge_index.shape[1]
    h = W0.shape[1]

    n_pad = _ceil_to(n + 1, NS * LN)
    rows_total = _ceil_to(-(-e // LN), NW * CH)
    ep = rows_total * LN

    pad_idx = jnp.full((ep - e,), n, jnp.int32)
    src2d = jnp.concatenate([edge_index[0], pad_idx]).reshape(rows_total, LN)
    dst2d = jnp.concatenate([edge_index[1], pad_idx]).reshape(rows_total, LN)
    xp = jnp.pad(x, ((0, n_pad - n), (0, 0)))
    w_head = jnp.concatenate([Wp, Wv, jnp.zeros((h, h - 2), jnp.float32)], axis=1)

    ones8 = jnp.ones((LN, HW), jnp.float32)
    zeros8 = jnp.zeros((LN, HW), jnp.float32)

    deg = _make_deg(n_pad, rows_total)(dst2d, ones8, zeros8)
    dis, xs = _prep_call(n, deg[0, :, :1], deg[1, :, :1], xp, W0)

    agg = _make_agg(n_pad, rows_total)

    for w_next, b_cur in ((W1, b0), (W2, b1), (W3, b2), (w_head, b3)):
        acc = agg(src2d, dst2d, xs, zeros8)
        xs = _dense_call(acc[0], acc[1], xs, dis, b_cur.reshape(1, -1), w_next)

    acc = agg(src2d, dst2d, xs, zeros8)
    proba_pad, value = _head_call(
        n, acc[0], acc[1], xs, dis, bp.reshape(1, 1), bv.reshape(1, 1)
    )
    return (proba_pad[:n], value)


# speedup vs baseline: 109.5537x; 1.6843x over previous
import functools

import jax
import jax.numpy as jnp
from jax import lax
from jax.experimental import pallas as pl
from jax.experimental.pallas import tpu as pltpu
from jax.experimental.pallas import tpu_sc as plsc

NC = 2
NS = 16
NW = NC * NS
LN = 128
CH = 8
HW = 8


def _sc_mesh():
    return plsc.VectorSubcoreMesh(
        core_axis_name="c", subcore_axis_name="s", num_cores=NC, num_subcores=NS
    )


_SC_PARAMS = pltpu.CompilerParams(use_tc_tiling_on_sc=False)


def _make_deg(n_pad, rows_total):
    ec = CH * LN
    nch = rows_total * LN // (NW * ec)
    D = 4
    zr = n_pad // NS
    zk = zr // LN

    @functools.partial(
        pl.kernel,
        out_type=jax.ShapeDtypeStruct((NC, n_pad, HW), jnp.float32),
        mesh=_sc_mesh(),
        compiler_params=_SC_PARAMS,
        scratch_types=[
            pltpu.VMEM_SHARED((n_pad, HW), jnp.float32),
            pltpu.VMEM((D, ec), jnp.int32),
            pltpu.VMEM((ec, HW), jnp.float32),
            pltpu.VMEM((LN, HW), jnp.float32),
            pltpu.SemaphoreType.DMA,
            pltpu.SemaphoreType.DMA,
        ],
    )
    def deg(dst_hbm, ones_hbm, zeros_hbm, acc_out,
            acc_sh, dst_v, ones_v, zbuf, sem_i, sem_s):
        c = lax.axis_index("c")
        s = lax.axis_index("s")
        pltpu.sync_copy(zeros_hbm, zbuf)
        pltpu.sync_copy(ones_hbm, ones_v)

        def zero_body(k, carry):
            pltpu.sync_copy(zbuf, acc_sh.at[pl.ds(s * zr + k * LN, LN)])
            return carry

        lax.fori_loop(0, zk, zero_body, 0)
        plsc.subcore_barrier()

        base = (c * NS + s) * nch * ec

        def fire_idx(q, slot):
            return pltpu.async_copy(dst_hbm.at[pl.ds(base + q * ec, ec)],
                                    dst_v.at[slot], sem_i)

        def wait_scatter(slot):
            pltpu.make_async_copy(ones_v, acc_sh.at[dst_v.at[slot]], sem_s).wait()

        for q in range(D - 1):
            fire_idx(q, q)

        def body(it, carry):
            for b in range(D):
                q = it * D + b

                @pl.when(jnp.logical_and(q >= 2, True))
                def _():
                    wait_scatter((b + D - 2) % D)

                @pl.when(q + D - 1 < nch)
                def _():
                    fire_idx(q + D - 1, (b + D - 1) % D)

                pltpu.make_async_copy(dst_hbm.at[pl.ds(base + q * ec, ec)],
                                      dst_v.at[b], sem_i).wait()
                pltpu.async_copy(ones_v, acc_sh.at[dst_v.at[b]], sem_s, add=True)
            return carry

        lax.fori_loop(0, nch // D, body, 0)
        wait_scatter((nch - 2) % D)
        wait_scatter((nch - 1) % D)
        plsc.subcore_barrier()
        pltpu.sync_copy(acc_sh.at[pl.ds(s * zr, zr)], acc_out.at[c, pl.ds(s * zr, zr)])

    return deg


def _make_agg(n_pad, rows_total):
    ec = CH * LN
    nch = rows_total * LN // (NW * ec)
    D = 7
    zr = n_pad // NS
    zk = zr // LN

    @functools.partial(
        pl.kernel,
        out_type=jax.ShapeDtypeStruct((NC, n_pad, HW), jnp.float32),
        mesh=_sc_mesh(),
        compiler_params=_SC_PARAMS,
        scratch_types=[
            pltpu.VMEM_SHARED((n_pad, HW), jnp.float32),
            pltpu.VMEM((D, ec), jnp.int32),
            pltpu.VMEM((D, ec), jnp.int32),
            pltpu.VMEM((D, ec, HW), jnp.float32),
            pltpu.VMEM((LN, HW), jnp.float32),
            pltpu.SemaphoreType.DMA,
            pltpu.SemaphoreType.DMA,
            pltpu.SemaphoreType.DMA,
        ],
    )
    def agg(src_hbm, dst_hbm, xs_hbm, zeros_hbm, acc_out,
            acc_sh, src_v, dst_v, rows_v, zbuf, sem_i, sem_g, sem_s):
        c = lax.axis_index("c")
        s = lax.axis_index("s")
        pltpu.sync_copy(zeros_hbm, zbuf)

        def zero_body(k, carry):
            pltpu.sync_copy(zbuf, acc_sh.at[pl.ds(s * zr + k * LN, LN)])
            return carry

        lax.fori_loop(0, zk, zero_body, 0)
        plsc.subcore_barrier()

        base = (c * NS + s) * nch * ec

        def fire_idx(q, slot):
            pltpu.async_copy(src_hbm.at[pl.ds(base + q * ec, ec)],
                             src_v.at[slot], sem_i)
            pltpu.async_copy(dst_hbm.at[pl.ds(base + q * ec, ec)],
                             dst_v.at[slot], sem_i)

        def wait_idx(q, slot):
            pltpu.make_async_copy(src_hbm.at[pl.ds(base + q * ec, ec)],
                                  src_v.at[slot], sem_i).wait()
            pltpu.make_async_copy(dst_hbm.at[pl.ds(base + q * ec, ec)],
                                  dst_v.at[slot], sem_i).wait()

        def fire_gather(slot):
            pltpu.async_copy(xs_hbm.at[src_v.at[slot]], rows_v.at[slot], sem_g)

        def wait_gather(slot):
            pltpu.make_async_copy(xs_hbm.at[src_v.at[slot]], rows_v.at[slot],
                                  sem_g).wait()

        def fire_scatter(slot):
            pltpu.async_copy(rows_v.at[slot], acc_sh.at[dst_v.at[slot]], sem_s,
                             add=True)

        def wait_scatter(slot):
            pltpu.make_async_copy(rows_v.at[slot], acc_sh.at[dst_v.at[slot]],
                                  sem_s).wait()

        for q in range(5):
            fire_idx(q, q)
        for q in range(3):
            wait_idx(q, q)
            fire_gather(q)

        def body(it, carry):
            for b in range(D):
                q = it * D + b

                @pl.when(q >= 2)
                def _():
                    wait_scatter((b + D - 2) % D)

                @pl.when(q + 5 < nch)
                def _():
                    fire_idx(q + 5, (b + 5) % D)

                @pl.when(q + 3 < nch)
                def _():
                    wait_idx(q + 3, (b + 3) % D)
                    fire_gather((b + 3) % D)

                wait_gather(b)
                fire_scatter(b)
            return carry

        lax.fori_loop(0, nch // D, body, 0)
        wait_scatter((nch - 2) % D)
        wait_scatter((nch - 1) % D)
        plsc.subcore_barrier()
        pltpu.sync_copy(acc_sh.at[pl.ds(s * zr, zr)], acc_out.at[c, pl.ds(s * zr, zr)])

    return agg


_TC_R = 2048


def _dot(a, b):
    return lax.dot_general(a, b, (((1,), (0,)), ((), ())),
                           precision=lax.Precision.DEFAULT,
                           preferred_element_type=jnp.float32)


def _elu(v):
    return jnp.where(v > 0.0, v, jnp.exp(jnp.minimum(v, 0.0)) - 1.0)


def _prep_call(n, deg0, deg1, xp, w0):
    n_pad, ndim = xp.shape
    g = n_pad // _TC_R

    def body(d0_ref, d1_ref, x_ref, w0_ref, dis_ref, xs_ref):
        i = pl.program_id(0)
        rows = i * _TC_R + lax.broadcasted_iota(jnp.int32, (_TC_R, 1), 0)
        deg = d0_ref[...] + d1_ref[...] + 1.0
        dis = jnp.where(rows < n, lax.rsqrt(deg), 0.0)
        dis_ref[...] = dis
        xs_ref[...] = dis * _dot(x_ref[...], w0_ref[...])

    col = lambda i: (i, 0)
    fixed = lambda i: (0, 0)
    return pl.pallas_call(
        body,
        grid=(g,),
        in_specs=[
            pl.BlockSpec((_TC_R, 1), col),
            pl.BlockSpec((_TC_R, 1), col),
            pl.BlockSpec((_TC_R, ndim), col),
            pl.BlockSpec((ndim, HW), fixed),
        ],
        out_specs=[
            pl.BlockSpec((_TC_R, 1), col),
            pl.BlockSpec((_TC_R, HW), col),
        ],
        out_shape=[
            jax.ShapeDtypeStruct((n_pad, 1), jnp.float32),
            jax.ShapeDtypeStruct((n_pad, HW), jnp.float32),
        ],
    )(deg0, deg1, xp, w0)


def _dense_call(acc0, acc1, xs, dis, b_row, w_next):
    n_pad = xs.shape[0]
    g = n_pad // _TC_R

    def body(a0_ref, a1_ref, xs_ref, dis_ref, b_ref, w_ref, out_ref):
        dis = dis_ref[...]
        h = _elu(dis * (a0_ref[...] + a1_ref[...] + xs_ref[...]) + b_ref[...])
        out_ref[...] = dis * _dot(h, w_ref[...])

    col = lambda i: (i, 0)
    fixed = lambda i: (0, 0)
    return pl.pallas_call(
        body,
        grid=(g,),
        in_specs=[
            pl.BlockSpec((_TC_R, HW), col),
            pl.BlockSpec((_TC_R, HW), col),
            pl.BlockSpec((_TC_R, HW), col),
            pl.BlockSpec((_TC_R, 1), col),
            pl.BlockSpec((1, HW), fixed),
            pl.BlockSpec((HW, HW), fixed),
        ],
        out_specs=pl.BlockSpec((_TC_R, HW), col),
        out_shape=jax.ShapeDtypeStruct((n_pad, HW), jnp.float32),
    )(acc0, acc1, xs, dis, b_row, w_next)


def _head_call(n, acc0, acc1, xs, dis, bp, bv):
    n_pad = xs.shape[0]
    g = n_pad // _TC_R

    def body(a0_ref, a1_ref, xs_ref, dis_ref, bp_ref, bv_ref,
             proba_ref, value_ref, vsum):
        i = pl.program_id(0)
        out = dis_ref[...] * (a0_ref[...] + a1_ref[...] + xs_ref[...])
        proba_ref[...] = out[:, 0:1] + bp_ref[...]

        @pl.when(i == 0)
        def _():
            vsum[...] = jnp.zeros_like(vsum)

        vsum[...] += jnp.sum(out[:, 1:2], axis=0, keepdims=True)

        @pl.when(i == g - 1)
        def _():
            value_ref[...] = vsum[...] * (1.0 / n) + bv_ref[...]

    col = lambda i: (i, 0)
    fixed = lambda i: (0, 0)
    return pl.pallas_call(
        body,
        grid=(g,),
        in_specs=[
            pl.BlockSpec((_TC_R, HW), col),
            pl.BlockSpec((_TC_R, HW), col),
            pl.BlockSpec((_TC_R, HW), col),
            pl.BlockSpec((_TC_R, 1), col),
            pl.BlockSpec((1, 1), fixed),
            pl.BlockSpec((1, 1), fixed),
        ],
        out_specs=[
            pl.BlockSpec((_TC_R, 1), col),
            pl.BlockSpec((1, 1), fixed),
        ],
        out_shape=[
            jax.ShapeDtypeStruct((n_pad, 1), jnp.float32),
            jax.ShapeDtypeStruct((1, 1), jnp.float32),
        ],
        scratch_shapes=[pltpu.VMEM((1, 1), jnp.float32)],
    )(acc0, acc1, xs, dis, bp, bv)


def _ceil_to(v, m):
    return -(-v // m) * m


def kernel(x, edge_index, W0, b0, W1, b1, W2, b2, W3, b3, Wv, bv, Wp, bp):
    n, ndim = x.shape
    e = edge_index.shape[1]
    h = W0.shape[1]

    n_pad = _ceil_to(n + 1, NS * LN)
    rows_total = _ceil_to(-(-e // LN), NW * CH)
    ep = rows_total * LN

    pad_idx = n + jnp.arange(ep - e, dtype=jnp.int32) % (n_pad - n)
    src2d = jnp.concatenate([edge_index[0], pad_idx])
    dst2d = jnp.concatenate([edge_index[1], pad_idx])
    xp = jnp.pad(x, ((0, n_pad - n), (0, 0)))
    w_head = jnp.concatenate([Wp, Wv, jnp.zeros((h, h - 2), jnp.float32)], axis=1)

    ones8 = jnp.ones((CH * LN, HW), jnp.float32)
    zeros8 = jnp.zeros((LN, HW), jnp.float32)

    deg = _make_deg(n_pad, rows_total)(dst2d, ones8, zeros8)
    dis, xs = _prep_call(n, deg[0, :, :1], deg[1, :, :1], xp, W0)

    agg = _make_agg(n_pad, rows_total)

    for w_next, b_cur in ((W1, b0), (W2, b1), (W3, b2), (w_head, b3)):
        acc = agg(src2d, dst2d, xs, zeros8)
        xs = _dense_call(acc[0], acc[1], xs, dis, b_cur.reshape(1, -1), w_next)

    acc = agg(src2d, dst2d, xs, zeros8)
    proba_pad, value = _head_call(
        n, acc[0], acc[1], xs, dis, bp.reshape(1, 1), bv.reshape(1, 1)
    )
    return (proba_pad[:n], value)
